# baseline (speedup 1.0000x reference)
import functools

import jax
import jax.numpy as jnp
from jax import lax
from jax.experimental import pallas as pl
from jax.experimental.pallas import tpu as pltpu
from jax.experimental.pallas import tpu_sc as plsc

VOCAB = 100000
EMBED_DIM = 64
N_OOV = 16
OOV_BASE = VOCAB - N_OOV

_info = plsc.get_sparse_core_info()
NC, NS, L = _info.num_cores, _info.num_subcores, _info.num_lanes
NW = NC * NS

BATCH = 4096
SEQ = 50
BBLK = BATCH // NW
N_CHUNKS = SEQ
GROUPS = BBLK // L
CT = EMBED_DIM // 8
NBUF = 5
DEPTH = 3
N_STEPS = N_CHUNKS // NBUF


def _body(xt_hbm, w2v_hbm, oov_hbm, out_hbm, idx_v, oov_v, rows_v, trans_v,
          sem_g, sem_w):
    wid = lax.axis_index("s") * NC + lax.axis_index("c")
    b0 = wid * BBLK
    pltpu.sync_copy(xt_hbm.at[:, pl.ds(b0, BBLK)], idx_v)
    pltpu.sync_copy(oov_hbm, oov_v)

    iota = lax.iota(jnp.int32, L)

    def fire_gather(g, b):
        pltpu.async_copy(w2v_hbm.at[idx_v.at[g]], rows_v.at[b], sem_g.at[b])

    def wait_gather(g, b):
        pltpu.make_async_copy(
            w2v_hbm.at[idx_v.at[g]], rows_v.at[b], sem_g.at[b]
        ).wait()

    def fire_write(g, b):
        pltpu.async_copy(trans_v.at[b], out_hbm.at[g, :, wid], sem_w.at[b])

    def wait_write(b):
        pltpu.make_async_copy(
            trans_v.at[b], out_hbm.at[0, :, wid], sem_w.at[b]
        ).wait()

    def patch_chunk(g, b):
        def group_body(q, _):
            idx_g = idx_v[g, pl.ds(q * L, L)]
            is_oov = idx_g >= OOV_BASE
            cnt = plsc.all_reduce_population_count(is_oov)

            @pl.when(cnt[0] > 0)
            def _patch():
                mapped = jnp.maximum(idx_g - OOV_BASE, 0)
                pos = q * L + iota
                buf = rows_v.at[b]

                def col_body(c, _):
                    cv = jnp.full((L,), 0, jnp.int32) + c
                    vals = plsc.load_gather(oov_v, [mapped, cv], mask=is_oov)
                    plsc.store_scatter(buf, [pos, cv], vals, mask=is_oov)
                    return 0

                lax.fori_loop(0, EMBED_DIM, col_body, 0)

            return 0

        lax.fori_loop(0, GROUPS, group_body, 0)

    pr_vecs = [(iota + d) & (L - 1) for d in range(L)]
    col_vecs = [iota + c0 for c0 in range(0, EMBED_DIM, L)]
    i0_vecs = [(iota + c0) >> 3 for c0 in range(0, EMBED_DIM, L)]
    h_vecs = [((iota + c0) & 7) * BBLK for c0 in range(0, EMBED_DIM, L)]

    def transpose_chunk(b):
        rows = rows_v.at[b]
        trans = trans_v.at[b]

        def jblk_body(jb, _):
            j0 = jb * L
            for d in range(L):
                row_idx = pr_vecs[d] + j0
                for c in range(EMBED_DIM // L):
                    vals = plsc.load_gather(rows, [row_idx, col_vecs[c]])
                    plsc.store_scatter(trans, [i0_vecs[c], h_vecs[c] + row_idx],
                                       vals)
            return 0

        lax.fori_loop(0, GROUPS, jblk_body, 0)

    for b in range(DEPTH):
        fire_gather(b, b)

    def step(s, _):
        for b in range(NBUF):
            g = s * NBUF + b

            wait_gather(g, b)
            patch_chunk(g, b)

            @pl.when(g >= NBUF)
            def _drain_prev_write():
                wait_write(b)

            transpose_chunk(b)
            fire_write(g, b)

            @pl.when(g + DEPTH < N_CHUNKS)
            def _fire_ahead():
                fire_gather(g + DEPTH, (b + DEPTH) % NBUF)

        return 0

    lax.fori_loop(0, N_STEPS, step, 0)
    for b in range(NBUF):
        wait_write(b)


@functools.partial(
    pl.kernel,
    out_type=jax.ShapeDtypeStruct((SEQ, CT, NW, 8 * BBLK), jnp.float32),
    mesh=plsc.VectorSubcoreMesh(core_axis_name="c", subcore_axis_name="s"),
    scratch_types=[
        pltpu.VMEM((SEQ, BBLK), jnp.int32),
        pltpu.VMEM((N_OOV, EMBED_DIM), jnp.float32),
        pltpu.VMEM((NBUF, BBLK, EMBED_DIM), jnp.float32),
        pltpu.VMEM((NBUF, CT, 8 * BBLK), jnp.float32),
        pltpu.SemaphoreType.DMA((NBUF,)),
        pltpu.SemaphoreType.DMA((NBUF,)),
    ],
    compiler_params=pltpu.CompilerParams(
        use_tc_tiling_on_sc=False, needs_layout_passes=False
    ),
)
def _lookup(xt_hbm, w2v_hbm, oov_hbm, out_hbm, idx_v, oov_v, rows_v, trans_v,
            sem_g, sem_w):
    _body(xt_hbm, w2v_hbm, oov_hbm, out_hbm, idx_v, oov_v, rows_v, trans_v,
          sem_g, sem_w)


@jax.jit
def kernel(x, w2v_mat, oov_table):
    xt = jnp.swapaxes(x, 0, 1).astype(jnp.int32)
    out = _lookup(xt, w2v_mat, oov_table)
    out = out.reshape(SEQ, CT, NW, 8, BBLK)
    out = out.transpose(2, 4, 0, 1, 3)
    return out.reshape(BATCH, SEQ, EMBED_DIM)


# speedup vs baseline: 1.5485x; 1.5485x over previous
import functools

import jax
import jax.numpy as jnp
from jax import lax
from jax.experimental import pallas as pl
from jax.experimental.pallas import tpu as pltpu
from jax.experimental.pallas import tpu_sc as plsc

VOCAB = 100000
EMBED_DIM = 64
N_OOV = 16
OOV_BASE = VOCAB - N_OOV

_info = plsc.get_sparse_core_info()
NC, NS, L = _info.num_cores, _info.num_subcores, _info.num_lanes
NW = NC * NS

BATCH = 4096
SEQ = 50
BBLK = BATCH // NW
N_CHUNKS = SEQ
GROUPS = BBLK // L
CT = EMBED_DIM // 8
NBUF = 5
DEPTH = 3
N_STEPS = N_CHUNKS // NBUF


def _body(xt_hbm, w2v_hbm, oov_hbm, out_hbm, idx_v, oov_v, rows_v, trans_v,
          sem_g, sem_w):
    wid = lax.axis_index("s") * NC + lax.axis_index("c")
    b0 = wid * BBLK
    pltpu.sync_copy(xt_hbm.at[:, pl.ds(b0, BBLK)], idx_v)
    pltpu.sync_copy(oov_hbm, oov_v)

    iota = lax.iota(jnp.int32, L)

    def fire_gather(g, b):
        pltpu.async_copy(w2v_hbm.at[idx_v.at[g]], rows_v.at[b], sem_g.at[b])

    def wait_gather(g, b):
        pltpu.make_async_copy(
            w2v_hbm.at[idx_v.at[g]], rows_v.at[b], sem_g.at[b]
        ).wait()

    def fire_write(g, b):
        pltpu.async_copy(trans_v.at[b], out_hbm.at[g, :, wid], sem_w.at[b])

    def wait_write(b):
        pltpu.make_async_copy(
            trans_v.at[b], out_hbm.at[0, :, wid], sem_w.at[b]
        ).wait()

    def patch_chunk(g, b):
        def group_body(q, _):
            idx_g = idx_v[g, pl.ds(q * L, L)]
            is_oov = idx_g >= OOV_BASE
            cnt = plsc.all_reduce_population_count(is_oov)

            @pl.when(cnt[0] > 0)
            def _patch():
                mapped = jnp.maximum(idx_g - OOV_BASE, 0)
                pos = q * L + iota
                buf = rows_v.at[b]

                def col_body(c, _):
                    cv = jnp.full((L,), 0, jnp.int32) + c
                    vals = plsc.load_gather(oov_v, [mapped, cv], mask=is_oov)
                    plsc.store_scatter(buf, [pos, cv], vals, mask=is_oov)
                    return 0

                lax.fori_loop(0, EMBED_DIM, col_body, 0)

            return 0

        lax.fori_loop(0, GROUPS, group_body, 0)

    pr_vecs = [(iota + d) & (L - 1) for d in range(L)]
    col_vecs = [iota + c0 for c0 in range(0, EMBED_DIM, L)]
    i0_vecs = [(iota + c0) >> 3 for c0 in range(0, EMBED_DIM, L)]
    h_vecs = [((iota + c0) & 7) * BBLK for c0 in range(0, EMBED_DIM, L)]

    def transpose_chunk(b):
        rows = rows_v.at[b]
        trans = trans_v.at[b]

        NCB = EMBED_DIM // L

        def jblk_body(jb, _):
            j0 = jb * L
            row_idxs = [pr_vecs[d] + j0 for d in range(L)]

            def gathers(d):
                return [
                    plsc.load_gather(rows, [row_idxs[d], col_vecs[c]])
                    for c in range(NCB)
                ]

            def scatters(d, vals):
                for c in range(NCB):
                    plsc.store_scatter(
                        trans, [i0_vecs[c], h_vecs[c] + row_idxs[d]], vals[c]
                    )

            vals = gathers(0)
            for d in range(1, L):
                nxt = gathers(d)
                scatters(d - 1, vals)
                vals = nxt
            scatters(L - 1, vals)
            return 0

        lax.fori_loop(0, GROUPS, jblk_body, 0)

    for b in range(DEPTH):
        fire_gather(b, b)

    def step(s, _):
        for b in range(NBUF):
            g = s * NBUF + b

            wait_gather(g, b)
            patch_chunk(g, b)

            @pl.when(g >= NBUF)
            def _drain_prev_write():
                wait_write(b)

            transpose_chunk(b)
            fire_write(g, b)

            @pl.when(g + DEPTH < N_CHUNKS)
            def _fire_ahead():
                fire_gather(g + DEPTH, (b + DEPTH) % NBUF)

        return 0

    lax.fori_loop(0, N_STEPS, step, 0)
    for b in range(NBUF):
        wait_write(b)


@functools.partial(
    pl.kernel,
    out_type=jax.ShapeDtypeStruct((SEQ, CT, NW, 8 * BBLK), jnp.float32),
    mesh=plsc.VectorSubcoreMesh(core_axis_name="c", subcore_axis_name="s"),
    scratch_types=[
        pltpu.VMEM((SEQ, BBLK), jnp.int32),
        pltpu.VMEM((N_OOV, EMBED_DIM), jnp.float32),
        pltpu.VMEM((NBUF, BBLK, EMBED_DIM), jnp.float32),
        pltpu.VMEM((NBUF, CT, 8 * BBLK), jnp.float32),
        pltpu.SemaphoreType.DMA((NBUF,)),
        pltpu.SemaphoreType.DMA((NBUF,)),
    ],
    compiler_params=pltpu.CompilerParams(
        use_tc_tiling_on_sc=False, needs_layout_passes=False
    ),
)
def _lookup(xt_hbm, w2v_hbm, oov_hbm, out_hbm, idx_v, oov_v, rows_v, trans_v,
            sem_g, sem_w):
    _body(xt_hbm, w2v_hbm, oov_hbm, out_hbm, idx_v, oov_v, rows_v, trans_v,
          sem_g, sem_w)


@jax.jit
def kernel(x, w2v_mat, oov_table):
    xt = jnp.swapaxes(x, 0, 1).astype(jnp.int32)
    out = _lookup(xt, w2v_mat, oov_table)
    out = out.reshape(SEQ, CT, NW, 8, BBLK)
    out = out.transpose(2, 4, 0, 1, 3)
    return out.reshape(BATCH, SEQ, EMBED_DIM)


# speedup vs baseline: 1.6321x; 1.0540x over previous
import functools

import jax
import jax.numpy as jnp
from jax import lax
from jax.experimental import pallas as pl
from jax.experimental.pallas import tpu as pltpu
from jax.experimental.pallas import tpu_sc as plsc

VOCAB = 100000
EMBED_DIM = 64
N_OOV = 16
OOV_BASE = VOCAB - N_OOV

_info = plsc.get_sparse_core_info()
NC, NS, L = _info.num_cores, _info.num_subcores, _info.num_lanes
NW = NC * NS

BATCH = 4096
SEQ = 50
BBLK = BATCH // NW
N_CHUNKS = SEQ
GROUPS = BBLK // L
CT = EMBED_DIM // 8
NBUF = 5
DEPTH = 4
N_STEPS = N_CHUNKS // NBUF


def _body(xt_hbm, w2v_hbm, oov_hbm, out_hbm, idx_v, oov_v, rows_v, trans_v,
          sem_g, sem_w):
    wid = lax.axis_index("s") * NC + lax.axis_index("c")
    b0 = wid * BBLK
    pltpu.sync_copy(xt_hbm.at[:, pl.ds(b0, BBLK)], idx_v)
    pltpu.sync_copy(oov_hbm, oov_v)

    iota = lax.iota(jnp.int32, L)

    def fire_gather(g, b):
        pltpu.async_copy(w2v_hbm.at[idx_v.at[g]], rows_v.at[b], sem_g.at[b])

    def wait_gather(g, b):
        pltpu.make_async_copy(
            w2v_hbm.at[idx_v.at[g]], rows_v.at[b], sem_g.at[b]
        ).wait()

    def fire_write(g, b):
        pltpu.async_copy(trans_v.at[b], out_hbm.at[g, :, wid], sem_w.at[b])

    def wait_write(b):
        pltpu.make_async_copy(
            trans_v.at[b], out_hbm.at[0, :, wid], sem_w.at[b]
        ).wait()

    def patch_chunk(g, b):
        def group_body(q, _):
            idx_g = idx_v[g, pl.ds(q * L, L)]
            is_oov = idx_g >= OOV_BASE
            cnt = plsc.all_reduce_population_count(is_oov)

            @pl.when(cnt[0] > 0)
            def _patch():
                mapped = jnp.maximum(idx_g - OOV_BASE, 0)
                pos = q * L + iota
                buf = rows_v.at[b]

                def col_body(c, _):
                    cv = jnp.full((L,), 0, jnp.int32) + c
                    vals = plsc.load_gather(oov_v, [mapped, cv], mask=is_oov)
                    plsc.store_scatter(buf, [pos, cv], vals, mask=is_oov)
                    return 0

                lax.fori_loop(0, EMBED_DIM, col_body, 0)

            return 0

        mx = idx_v[g, pl.ds(0, L)]
        for q in range(1, GROUPS):
            mx = jnp.maximum(mx, idx_v[g, pl.ds(q * L, L)])
        any_oov = plsc.all_reduce_population_count(mx >= OOV_BASE)

        @pl.when(any_oov[0] > 0)
        def _patch_chunk():
            lax.fori_loop(0, GROUPS, group_body, 0)

    pr_vecs = [(iota + d) & (L - 1) for d in range(L)]
    col_vecs = [iota + c0 for c0 in range(0, EMBED_DIM, L)]
    i0_vecs = [(iota + c0) >> 3 for c0 in range(0, EMBED_DIM, L)]
    h_vecs = [((iota + c0) & 7) * BBLK for c0 in range(0, EMBED_DIM, L)]

    def transpose_chunk(b):
        rows = rows_v.at[b]
        trans = trans_v.at[b]

        NCB = EMBED_DIM // L

        def jblk_body(jb, _):
            j0 = jb * L
            row_idxs = [pr_vecs[d] + j0 for d in range(L)]

            def gathers(d):
                return [
                    plsc.load_gather(rows, [row_idxs[d], col_vecs[c]])
                    for c in range(NCB)
                ]

            def scatters(d, vals):
                for c in range(NCB):
                    plsc.store_scatter(
                        trans, [i0_vecs[c], h_vecs[c] + row_idxs[d]], vals[c]
                    )

            vals = gathers(0)
            for d in range(1, L):
                nxt = gathers(d)
                scatters(d - 1, vals)
                vals = nxt
            scatters(L - 1, vals)
            return 0

        lax.fori_loop(0, GROUPS, jblk_body, 0)

    for b in range(DEPTH):
        fire_gather(b, b)

    def step(s, _):
        for b in range(NBUF):
            g = s * NBUF + b

            wait_gather(g, b)
            patch_chunk(g, b)

            @pl.when(g >= NBUF)
            def _drain_prev_write():
                wait_write(b)

            transpose_chunk(b)
            fire_write(g, b)

            @pl.when(g + DEPTH < N_CHUNKS)
            def _fire_ahead():
                fire_gather(g + DEPTH, (b + DEPTH) % NBUF)

        return 0

    lax.fori_loop(0, N_STEPS, step, 0)
    for b in range(NBUF):
        wait_write(b)


@functools.partial(
    pl.kernel,
    out_type=jax.ShapeDtypeStruct((SEQ, CT, NW, 8 * BBLK), jnp.float32),
    mesh=plsc.VectorSubcoreMesh(core_axis_name="c", subcore_axis_name="s"),
    scratch_types=[
        pltpu.VMEM((SEQ, BBLK), jnp.int32),
        pltpu.VMEM((N_OOV, EMBED_DIM), jnp.float32),
        pltpu.VMEM((NBUF, BBLK, EMBED_DIM), jnp.float32),
        pltpu.VMEM((NBUF, CT, 8 * BBLK), jnp.float32),
        pltpu.SemaphoreType.DMA((NBUF,)),
        pltpu.SemaphoreType.DMA((NBUF,)),
    ],
    compiler_params=pltpu.CompilerParams(
        use_tc_tiling_on_sc=False, needs_layout_passes=False
    ),
)
def _lookup(xt_hbm, w2v_hbm, oov_hbm, out_hbm, idx_v, oov_v, rows_v, trans_v,
            sem_g, sem_w):
    _body(xt_hbm, w2v_hbm, oov_hbm, out_hbm, idx_v, oov_v, rows_v, trans_v,
          sem_g, sem_w)


@jax.jit
def kernel(x, w2v_mat, oov_table):
    xt = jnp.swapaxes(x, 0, 1).astype(jnp.int32)
    out = _lookup(xt, w2v_mat, oov_table)
    out = out.reshape(SEQ, CT, NW, 8, BBLK)
    out = out.transpose(2, 4, 0, 1, 3)
    return out.reshape(BATCH, SEQ, EMBED_DIM)
